# Initial kernel scaffold; baseline (speedup 1.0000x reference)
import jax
import jax.numpy as jnp
from jax.experimental import pallas as pl


def kernel(x, batch, params):
    raise NotImplementedError("write your pallas kernel here")


# speedup vs baseline: 43.6035x; 43.6035x over previous
import jax
import jax.numpy as jnp
import numpy as np
from jax.experimental import pallas as pl

_G, _P, _K = 256, 64, 16
_NF = 7
_KS = [64, 128, 256]
_BG = 16
_BN_EPS = 1e-5
_DIMS = [_NF, _NF + _KS[0], _NF + _KS[0] + _KS[1]]
_IMAX = np.int32(0x7FFFFFFF)


def _dot(a, b):
    return jnp.dot(a, b, preferred_element_type=jnp.float32)


def _edge_layer(h, c, A, B, b1, W2, b2, W3, b3, eye_til):
    ls = A.shape[1]

    sq = jnp.sum(c * c, axis=1, keepdims=True)
    scoreTs = []
    for g in range(_BG):
        sl = slice(g * _P, (g + 1) * _P)
        cg = c[sl]
        cross = _dot(cg, cg.T)
        scoreTs.append(sq[sl] - 2.0 * cross)
    scoreT = jnp.concatenate(scoreTs, axis=1)

    jj = jax.lax.broadcasted_iota(jnp.int32, scoreT.shape, 0)
    ii = jax.lax.broadcasted_iota(jnp.int32, scoreT.shape, 1)
    ib = jax.lax.bitcast_convert_type(scoreT, jnp.int32)
    key = jnp.where(ib >= 0, ib, ib ^ 0x7FFFFFFF)
    key = jnp.bitwise_or(jnp.bitwise_and(key, ~63), jj)
    key = jnp.where(jj == jnp.bitwise_and(ii, _P - 1), _IMAX, key)

    ohs = []
    for _ in range(_K):
        m = jnp.min(key, axis=0, keepdims=True)
        oh = key == m
        ohs.append(jnp.where(oh, 1.0, 0.0))
        key = jnp.where(oh, _IMAX, key)

    u = _dot(h, A)
    v = _dot(h, B)

    h1s = []
    for g in range(_BG):
        sl = slice(g * _P, (g + 1) * _P)
        ot = jnp.concatenate([ohs[k][:, sl] for k in range(_K)], axis=1)
        cmat = jnp.concatenate([ot, eye_til], axis=0)
        rhs = jnp.concatenate([v[sl], u[sl]], axis=0)
        h1s.append(jnp.maximum(jax.lax.dot_general(
            cmat, rhs, (((0,), (0,)), ((), ())),
            preferred_element_type=jnp.float32) + b1, 0.0))
    h1 = jnp.concatenate(h1s, axis=0)

    h2 = jnp.maximum(_dot(h1, W2) + b2, 0.0)
    h3 = jnp.maximum(_dot(h2, W3) + b3, 0.0)

    m3 = h3.reshape(_BG, _K, _P, ls).sum(axis=1)
    return jnp.concatenate([m3.reshape(_BG * _P, ls), h], axis=1)


def _body(*args):
    out_ref = args[-1]
    x_ref = args[0]
    w = args[1:-1]
    h = x_ref[...]
    ei = jax.lax.broadcasted_iota(jnp.int32, (_P, _K * _P), 0)
    ee = jax.lax.broadcasted_iota(jnp.int32, (_P, _K * _P), 1)
    eye_til = jnp.where(jnp.bitwise_and(ee, _P - 1) == ei, 1.0, 0.0)
    for i in range(3):
        A, B, b1, W2, b2, W3, b3 = (r[...] for r in w[i * 7:(i + 1) * 7])
        c = h[:, :2] if i == 0 else h
        h = _edge_layer(h, c, A, B, b1, W2, b2, W3, b3, eye_til)
    fc1t, fc1b, fc2t, fc2b = (r[...] for r in w[21:25])

    pooled = h.reshape(_BG, _P, h.shape[1]).sum(axis=1)
    z = jnp.maximum(_dot(pooled, fc1t) + fc1b, 0.0)
    out_ref[...] = (_dot(z, fc2t) + fc2b).reshape(1, _BG, 128)


def _prep_weights(params):
    inv = np.float32(1.0 / np.sqrt(1.0 + _BN_EPS))
    ws = []
    for i, name in enumerate(('ec0', 'ec1', 'ec2')):
        p = params[name]
        folded = []
        for j in range(3):
            s = p['g%d' % j] * inv
            wt = p['W%d' % j].T * s[None, :]
            bf = p['b%d' % j] * s + p['beta%d' % j]
            folded.append((wt.astype(jnp.float32), bf.astype(jnp.float32)))
        d = _DIMS[i]
        wt0, b0 = folded[0]
        wa, wb = wt0[:d], wt0[d:]
        ws += [wa - wb, wb, b0[None, :],
               folded[1][0], folded[1][1][None, :],
               folded[2][0] * (1.0 / _K), folded[2][1][None, :] * (1.0 / _K)]
    fc1t = params['fc1_W'].T.astype(jnp.float32) * (1.0 / _P)
    fc1b = params['fc1_b'][None, :].astype(jnp.float32)
    fc2t = jnp.zeros((256, 128), jnp.float32).at[:, :5].set(params['fc2_W'].T)
    fc2b = jnp.zeros((1, 128), jnp.float32).at[:, :5].set(params['fc2_b'])
    ws += [fc1t, fc1b, fc2t, fc2b]
    return ws


def kernel(x, batch, params):
    del batch
    ws = _prep_weights(params)
    ins = [x] + ws
    in_specs = [pl.BlockSpec((_BG * _P, _NF), lambda c: (c, 0))]
    for a in ins[1:]:
        in_specs.append(pl.BlockSpec(a.shape, lambda c: (0, 0)))
    out = pl.pallas_call(
        _body,
        grid=(_G // _BG,),
        in_specs=in_specs,
        out_specs=pl.BlockSpec((1, _BG, 128), lambda c: (c, 0, 0)),
        out_shape=jax.ShapeDtypeStruct((_G // _BG, _BG, 128), jnp.float32),
    )(*ins)
    return out.reshape(_G, 128)[:, :5]
